# baseline (speedup 1.0000x reference)
import functools

import jax
import jax.numpy as jnp
from jax import lax
from jax.experimental import pallas as pl
from jax.experimental.pallas import tpu as pltpu
from jax.experimental.pallas import tpu_sc as plsc

NC = 2
NS = 16
NW = NC * NS


def _silu(v):
    return v * jax.nn.sigmoid(v)


def _pre_body(h_ref, w1s_ref, w1d_ref, be1_ref, hs_ref, hd_ref):
    hb = h_ref[...]
    hs_ref[...] = jnp.dot(hb, w1s_ref[...],
                          preferred_element_type=jnp.float32).astype(jnp.bfloat16)
    hd_ref[...] = (jnp.dot(hb, w1d_ref[...], preferred_element_type=jnp.float32)
                   + be1_ref[...]).astype(jnp.bfloat16)


def _tc_pre(h, w1s, w1d, be1, *, interpret=False):
    n, dh = h.shape
    bn = 2000
    grid = (n // bn,)
    return pl.pallas_call(
        _pre_body,
        grid=grid,
        in_specs=[
            pl.BlockSpec((bn, dh), lambda i: (i, 0)),
            pl.BlockSpec((dh, dh), lambda i: (0, 0)),
            pl.BlockSpec((dh, dh), lambda i: (0, 0)),
            pl.BlockSpec((1, dh), lambda i: (0, 0)),
        ],
        out_specs=[
            pl.BlockSpec((bn, dh), lambda i: (i, 0)),
            pl.BlockSpec((bn, dh), lambda i: (i, 0)),
        ],
        out_shape=[
            jax.ShapeDtypeStruct((n, dh), jnp.bfloat16),
            jax.ShapeDtypeStruct((n, dh), jnp.bfloat16),
        ],
        interpret=interpret,
    )(h, w1s, w1d, be1)


def _sc_gather(hs, hd, x16, src, dst):
    n, dh = hs.shape
    e = src.shape[0]
    pt = e // NW
    G = 400
    SUB = 80
    nsub = G // SUB
    nchunk = pt // G
    assert pt % G == 0 and G % SUB == 0

    mesh = plsc.VectorSubcoreMesh(core_axis_name="c", subcore_axis_name="s",
                                  num_cores=NC, num_subcores=NS)

    @functools.partial(
        pl.kernel,
        out_type=(
            jax.ShapeDtypeStruct((e, dh), jnp.bfloat16),
            jax.ShapeDtypeStruct((e, dh), jnp.bfloat16),
            jax.ShapeDtypeStruct((e, 16), jnp.float32),
            jax.ShapeDtypeStruct((e, 16), jnp.float32),
        ),
        mesh=mesh,
        scratch_types=[
            pltpu.VMEM((G,), jnp.int32),
            pltpu.VMEM((G,), jnp.int32),
            pltpu.VMEM((G, dh), jnp.bfloat16),
            pltpu.VMEM((G, dh), jnp.bfloat16),
            pltpu.VMEM((G, 16), jnp.float32),
            pltpu.VMEM((G, 16), jnp.float32),
            pltpu.SemaphoreType.DMA,
        ],
        compiler_params=pltpu.CompilerParams(use_tc_tiling_on_sc=False),
    )
    def gather_kernel(hs_hbm, hd_hbm, x_hbm, src_hbm, dst_hbm,
                      g1_hbm, g2_hbm, xs_hbm, xd_hbm,
                      sidx_v, didx_v, g1_v, g2_v, xs_v, xd_v, sem):
        wid = lax.axis_index("s") * NC + lax.axis_index("c")
        base = wid * pt

        def chunk(c, carry):
            off = base + c * G
            pltpu.sync_copy(src_hbm.at[pl.ds(off, G)], sidx_v)
            pltpu.sync_copy(dst_hbm.at[pl.ds(off, G)], didx_v)
            cps = []
            for s in range(nsub):
                si = sidx_v.at[pl.ds(s * SUB, SUB)]
                di = didx_v.at[pl.ds(s * SUB, SUB)]
                cps.append(pltpu.async_copy(
                    hs_hbm.at[si], g1_v.at[pl.ds(s * SUB, SUB)], sem))
                cps.append(pltpu.async_copy(
                    hd_hbm.at[di], g2_v.at[pl.ds(s * SUB, SUB)], sem))
                cps.append(pltpu.async_copy(
                    x_hbm.at[si], xs_v.at[pl.ds(s * SUB, SUB)], sem))
                cps.append(pltpu.async_copy(
                    x_hbm.at[di], xd_v.at[pl.ds(s * SUB, SUB)], sem))
            for cp in cps:
                cp.wait()
            pltpu.sync_copy(g1_v, g1_hbm.at[pl.ds(off, G)])
            pltpu.sync_copy(g2_v, g2_hbm.at[pl.ds(off, G)])
            pltpu.sync_copy(xs_v, xs_hbm.at[pl.ds(off, G)])
            pltpu.sync_copy(xd_v, xd_hbm.at[pl.ds(off, G)])
            return carry

        lax.fori_loop(0, nchunk, chunk, 0)

    return gather_kernel(hs, hd, x16, src, dst)


def _edge_body(g1_ref, g2_ref, xs_ref, xd_ref, ea_ref,
               w1e_ref, wsq_ref, we2_ref, be2_ref,
               wx1_ref, bx1_ref, wx2r_ref, bx2_ref,
               msgx_ref, wv_ref):
    d16 = xs_ref[...] - xd_ref[...]
    sq = jnp.sum(d16 * d16, axis=-1, keepdims=True)
    pre = (g1_ref[...].astype(jnp.float32) + g2_ref[...].astype(jnp.float32)
           + sq * wsq_ref[...]
           + jnp.dot(ea_ref[...], w1e_ref[...], preferred_element_type=jnp.float32))
    m1 = _silu(pre)
    msg = _silu(jnp.dot(m1, we2_ref[...], preferred_element_type=jnp.float32)
                + be2_ref[...])
    t = _silu(jnp.dot(msg, wx1_ref[...], preferred_element_type=jnp.float32)
              + bx1_ref[...])
    w = jnp.sum(t * wx2r_ref[...], axis=-1, keepdims=True) + bx2_ref[0, 0]
    b = msg.shape[0]
    lane = lax.broadcasted_iota(jnp.int32, (b, 16), 1)
    cnt16 = jnp.where(lane == 0, 1.0, 0.0).astype(jnp.float32)
    msgx_ref[...] = jnp.concatenate([msg, cnt16], axis=-1)
    wv_ref[...] = d16 * w


def _tc_edge(g1, g2, xs, xd, ea, w1e, wsq, we2, be2, wx1, bx1, wx2r, bx2,
             *, interpret=False):
    e, dh = g1.shape
    de = ea.shape[1]
    be = 2000
    grid = (e // be,)
    wspec = lambda shape: pl.BlockSpec(shape, lambda i: tuple(0 for _ in shape))
    return pl.pallas_call(
        _edge_body,
        grid=grid,
        in_specs=[
            pl.BlockSpec((be, dh), lambda i: (i, 0)),
            pl.BlockSpec((be, dh), lambda i: (i, 0)),
            pl.BlockSpec((be, 16), lambda i: (i, 0)),
            pl.BlockSpec((be, 16), lambda i: (i, 0)),
            pl.BlockSpec((be, de), lambda i: (i, 0)),
            wspec((de, dh)),
            wspec((1, dh)),
            wspec((dh, dh)),
            wspec((1, dh)),
            wspec((dh, dh)),
            wspec((1, dh)),
            wspec((1, dh)),
            wspec((1, 1)),
        ],
        out_specs=[
            pl.BlockSpec((be, dh + 16), lambda i: (i, 0)),
            pl.BlockSpec((be, 16), lambda i: (i, 0)),
        ],
        out_shape=[
            jax.ShapeDtypeStruct((e, dh + 16), jnp.float32),
            jax.ShapeDtypeStruct((e, 16), jnp.float32),
        ],
        interpret=interpret,
    )(g1, g2, xs, xd, ea, w1e, wsq, we2, be2, wx1, bx1, wx2r, bx2)


def _sc_scatter(msgx, wv, dst, src, n):
    e, dm = msgx.shape
    pt = e // NW
    G = 80
    nchunk = pt // G
    rz = n // NS
    ZB = 25
    nz = rz // ZB
    assert pt % G == 0 and rz % ZB == 0

    mesh = plsc.VectorSubcoreMesh(core_axis_name="c", subcore_axis_name="s",
                                  num_cores=NC, num_subcores=NS)

    @functools.partial(
        pl.kernel,
        out_type=(
            jax.ShapeDtypeStruct((NC * n, dm), jnp.float32),
            jax.ShapeDtypeStruct((NC * n, 16), jnp.float32),
        ),
        mesh=mesh,
        scratch_types=[
            pltpu.VMEM((G,), jnp.int32),
            pltpu.VMEM((G,), jnp.int32),
            pltpu.VMEM((G, dm), jnp.float32),
            pltpu.VMEM((G, 16), jnp.float32),
            pltpu.VMEM((ZB, dm), jnp.float32),
            pltpu.VMEM((ZB, 16), jnp.float32),
            pltpu.VMEM_SHARED((n, dm), jnp.float32),
            pltpu.VMEM_SHARED((n, 16), jnp.float32),
        ],
        compiler_params=pltpu.CompilerParams(use_tc_tiling_on_sc=False),
    )
    def scatter_kernel(msgx_hbm, wv_hbm, dst_hbm, src_hbm,
                       pd_hbm, ps_hbm,
                       didx_v, sidx_v, msg_v, wv_v, zbd_v, zbs_v,
                       acc_d, acc_s):
        cid = lax.axis_index("c")
        sid = lax.axis_index("s")
        wid = sid * NC + cid
        base = wid * pt
        r0 = sid * rz

        def zrow_d(i, carry):
            r = i // (dm // 16)
            c = (i % (dm // 16)) * 16
            zbd_v[r, pl.ds(c, 16)] = jnp.zeros((16,), jnp.float32)
            return carry
        lax.fori_loop(0, ZB * (dm // 16), zrow_d, 0)

        def zrow_s(i, carry):
            zbs_v[i, :] = jnp.zeros((16,), jnp.float32)
            return carry
        lax.fori_loop(0, ZB, zrow_s, 0)

        def zinit(j, carry):
            pltpu.sync_copy(zbd_v, acc_d.at[pl.ds(r0 + j * ZB, ZB)])
            pltpu.sync_copy(zbs_v, acc_s.at[pl.ds(r0 + j * ZB, ZB)])
            return carry
        lax.fori_loop(0, nz, zinit, 0)
        plsc.subcore_barrier()

        def chunk(c, carry):
            off = base + c * G
            pltpu.sync_copy(dst_hbm.at[pl.ds(off, G)], didx_v)
            pltpu.sync_copy(src_hbm.at[pl.ds(off, G)], sidx_v)
            pltpu.sync_copy(msgx_hbm.at[pl.ds(off, G)], msg_v)
            pltpu.sync_copy(wv_hbm.at[pl.ds(off, G)], wv_v)
            pltpu.sync_copy(msg_v, acc_d.at[didx_v], add=True)
            pltpu.sync_copy(wv_v, acc_s.at[sidx_v], add=True)
            return carry

        lax.fori_loop(0, nchunk, chunk, 0)
        plsc.subcore_barrier()

        def dump(j, carry):
            rr = r0 + j * ZB
            pltpu.sync_copy(acc_d.at[pl.ds(rr, ZB)], zbd_v)
            pltpu.sync_copy(zbd_v, pd_hbm.at[pl.ds(cid * n + rr, ZB)])
            pltpu.sync_copy(acc_s.at[pl.ds(rr, ZB)], zbs_v)
            pltpu.sync_copy(zbs_v, ps_hbm.at[pl.ds(cid * n + rr, ZB)])
            return carry
        lax.fori_loop(0, nz, dump, 0)

    return scatter_kernel(msgx, wv, dst, src)


def _post_body(pd0_ref, pd1_ref, ps0_ref, ps1_ref, h_ref, x16_ref,
               wh1a_ref, wh1b_ref, bh1_ref, wh2_ref, bh2_ref,
               lng_ref, lnb_ref, hn_ref, xn_ref):
    acc = pd0_ref[0] + pd1_ref[0]
    dh = h_ref.shape[1]
    agg_s = acc[:, :dh]
    cnt = jnp.maximum(acc[:, dh:dh + 1], 1.0)
    agg = agg_s / cnt
    hb = h_ref[...]
    t = _silu(jnp.dot(hb, wh1a_ref[...], preferred_element_type=jnp.float32)
              + jnp.dot(agg, wh1b_ref[...], preferred_element_type=jnp.float32)
              + bh1_ref[...])
    ph = jnp.dot(t, wh2_ref[...], preferred_element_type=jnp.float32) + bh2_ref[...]
    pre = hb + ph
    mu = jnp.mean(pre, axis=-1, keepdims=True)
    var = jnp.mean((pre - mu) ** 2, axis=-1, keepdims=True)
    hn_ref[...] = (pre - mu) * jax.lax.rsqrt(var + 1e-5) * lng_ref[...] + lnb_ref[...]
    ps = ps0_ref[0] + ps1_ref[0]
    xn_ref[...] = x16_ref[...] + ps / cnt


def _tc_post(pd, ps, h, x16, wh1a, wh1b, bh1, wh2, bh2, lng, lnb,
             *, interpret=False):
    n, dh = h.shape
    dm = pd.shape[1]
    bn = 2000
    grid = (n // bn,)
    wspec = lambda shape: pl.BlockSpec(shape, lambda i: tuple(0 for _ in shape))
    return pl.pallas_call(
        _post_body,
        grid=grid,
        in_specs=[
            pl.BlockSpec((1, bn, dm), lambda i: (0, i, 0)),
            pl.BlockSpec((1, bn, dm), lambda i: (1, i, 0)),
            pl.BlockSpec((1, bn, 16), lambda i: (0, i, 0)),
            pl.BlockSpec((1, bn, 16), lambda i: (1, i, 0)),
            pl.BlockSpec((bn, dh), lambda i: (i, 0)),
            pl.BlockSpec((bn, 16), lambda i: (i, 0)),
            wspec((dh, dh)),
            wspec((dh, dh)),
            wspec((1, dh)),
            wspec((dh, dh)),
            wspec((1, dh)),
            wspec((1, dh)),
            wspec((1, dh)),
        ],
        out_specs=[
            pl.BlockSpec((bn, dh), lambda i: (i, 0)),
            pl.BlockSpec((bn, 16), lambda i: (i, 0)),
        ],
        out_shape=[
            jax.ShapeDtypeStruct((n, dh), jnp.float32),
            jax.ShapeDtypeStruct((n, 16), jnp.float32),
        ],
        interpret=interpret,
    )(pd.reshape(2, n, dm), pd.reshape(2, n, dm),
      ps.reshape(2, n, 16), ps.reshape(2, n, 16), h, x16,
      wh1a, wh1b, bh1, wh2, bh2, lng, lnb)


def kernel(h, x, edge_index, edge_attr, W_e1, b_e1, W_e2, b_e2,
           W_h1, b_h1, W_h2, b_h2, W_x1, b_x1, W_x2, b_x2, ln_g, ln_b):
    n, dh = h.shape
    e = edge_index.shape[1]
    de = edge_attr.shape[1]
    dm = W_e2.shape[0]

    src = edge_index[0]
    dst = edge_index[1]
    x16 = jnp.pad(x, ((0, 0), (0, 16 - x.shape[1])))

    w1s = W_e1[:dh]
    w1d = W_e1[dh:2 * dh]
    wsq = W_e1[2 * dh:2 * dh + 1]
    w1e = W_e1[2 * dh + 1:]

    hs, hd = _tc_pre(h, w1s, w1d, b_e1.reshape(1, dm))
    g1, g2, xs, xd = _sc_gather(hs, hd, x16, src, dst)
    msgx, wv = _tc_edge(g1, g2, xs, xd, edge_attr,
                        w1e, wsq, W_e2, b_e2.reshape(1, dm),
                        W_x1, b_x1.reshape(1, dm),
                        W_x2.reshape(1, dm), b_x2.reshape(1, 1))
    pd, ps = _sc_scatter(msgx, wv, dst, src, n)
    hn, xn16 = _tc_post(pd, ps, h, x16,
                        W_h1[:dh], W_h1[dh:], b_h1.reshape(1, dh),
                        W_h2, b_h2.reshape(1, dh),
                        ln_g.reshape(1, dh), ln_b.reshape(1, dh))
    return (hn, xn16[:, :x.shape[1]])


# speedup vs baseline: 1.9090x; 1.9090x over previous
import functools

import jax
import jax.numpy as jnp
from jax import lax
from jax.experimental import pallas as pl
from jax.experimental.pallas import tpu as pltpu
from jax.experimental.pallas import tpu_sc as plsc

NC = 2
NS = 16
NW = NC * NS


def _silu(v):
    return v * jax.nn.sigmoid(v)


def _pre_body(h_ref, w1s_ref, w1d_ref, be1_ref, hs_ref, hd_ref):
    hb = h_ref[...]
    hs_ref[...] = jnp.dot(hb, w1s_ref[...], preferred_element_type=jnp.float32)
    hd_ref[...] = (jnp.dot(hb, w1d_ref[...], preferred_element_type=jnp.float32)
                   + be1_ref[...])


def _tc_pre(h, w1s, w1d, be1, *, interpret=False):
    n, dh = h.shape
    bn = 2000
    grid = (n // bn,)
    return pl.pallas_call(
        _pre_body,
        grid=grid,
        in_specs=[
            pl.BlockSpec((bn, dh), lambda i: (i, 0)),
            pl.BlockSpec((dh, dh), lambda i: (0, 0)),
            pl.BlockSpec((dh, dh), lambda i: (0, 0)),
            pl.BlockSpec((1, dh), lambda i: (0, 0)),
        ],
        out_specs=[
            pl.BlockSpec((bn, dh), lambda i: (i, 0)),
            pl.BlockSpec((bn, dh), lambda i: (i, 0)),
        ],
        out_shape=[
            jax.ShapeDtypeStruct((n, dh), jnp.float32),
            jax.ShapeDtypeStruct((n, dh), jnp.float32),
        ],
        interpret=interpret,
    )(h, w1s, w1d, be1)


def _sc_gather(hs, hd, x16, src, dst):
    n, dh = hs.shape
    e = src.shape[0]
    pt = e // NW
    G = 400
    SUB = 80
    nsub = G // SUB
    nchunk = pt // G
    assert pt % G == 0 and G % SUB == 0

    mesh = plsc.VectorSubcoreMesh(core_axis_name="c", subcore_axis_name="s",
                                  num_cores=NC, num_subcores=NS)

    @functools.partial(
        pl.kernel,
        out_type=(
            jax.ShapeDtypeStruct((e, dh), jnp.float32),
            jax.ShapeDtypeStruct((e, dh), jnp.float32),
            jax.ShapeDtypeStruct((e, 16), jnp.float32),
            jax.ShapeDtypeStruct((e, 16), jnp.float32),
        ),
        mesh=mesh,
        scratch_types=[
            pltpu.VMEM((G,), jnp.int32),
            pltpu.VMEM((G,), jnp.int32),
            pltpu.VMEM((G, dh), jnp.float32),
            pltpu.VMEM((G, dh), jnp.float32),
            pltpu.VMEM((G, 16), jnp.float32),
            pltpu.VMEM((G, 16), jnp.float32),
            pltpu.SemaphoreType.DMA,
        ],
        compiler_params=pltpu.CompilerParams(use_tc_tiling_on_sc=False),
    )
    def gather_kernel(hs_hbm, hd_hbm, x_hbm, src_hbm, dst_hbm,
                      g1_hbm, g2_hbm, xs_hbm, xd_hbm,
                      sidx_v, didx_v, g1_v, g2_v, xs_v, xd_v, sem):
        wid = lax.axis_index("s") * NC + lax.axis_index("c")
        base = wid * pt

        def chunk(c, carry):
            off = base + c * G
            pltpu.sync_copy(src_hbm.at[pl.ds(off, G)], sidx_v)
            pltpu.sync_copy(dst_hbm.at[pl.ds(off, G)], didx_v)
            cps = []
            for s in range(nsub):
                si = sidx_v.at[pl.ds(s * SUB, SUB)]
                di = didx_v.at[pl.ds(s * SUB, SUB)]
                cps.append(pltpu.async_copy(
                    hs_hbm.at[si], g1_v.at[pl.ds(s * SUB, SUB)], sem))
                cps.append(pltpu.async_copy(
                    hd_hbm.at[di], g2_v.at[pl.ds(s * SUB, SUB)], sem))
                cps.append(pltpu.async_copy(
                    x_hbm.at[si], xs_v.at[pl.ds(s * SUB, SUB)], sem))
                cps.append(pltpu.async_copy(
                    x_hbm.at[di], xd_v.at[pl.ds(s * SUB, SUB)], sem))
            for cp in cps:
                cp.wait()
            pltpu.sync_copy(g1_v, g1_hbm.at[pl.ds(off, G)])
            pltpu.sync_copy(g2_v, g2_hbm.at[pl.ds(off, G)])
            pltpu.sync_copy(xs_v, xs_hbm.at[pl.ds(off, G)])
            pltpu.sync_copy(xd_v, xd_hbm.at[pl.ds(off, G)])
            return carry

        lax.fori_loop(0, nchunk, chunk, 0)

    return gather_kernel(hs, hd, x16, src, dst)


def _edge_body(g1_ref, g2_ref, xs_ref, xd_ref, ea_ref,
               w1e_ref, wsq_ref, we2_ref, be2_ref,
               wx1_ref, bx1_ref, wx2r_ref, bx2_ref,
               msg_ref, wv_ref):
    d16 = xs_ref[...] - xd_ref[...]
    sq = jnp.sum(d16 * d16, axis=-1, keepdims=True)
    pre = (g1_ref[...] + g2_ref[...]
           + sq * wsq_ref[...]
           + jnp.dot(ea_ref[...], w1e_ref[...], preferred_element_type=jnp.float32))
    m1 = _silu(pre).astype(jnp.bfloat16)
    msg = _silu(jnp.dot(m1, we2_ref[...], preferred_element_type=jnp.float32)
                + be2_ref[...])
    msgb = msg.astype(jnp.bfloat16)
    t = _silu(jnp.dot(msgb, wx1_ref[...], preferred_element_type=jnp.float32)
              + bx1_ref[...])
    w = jnp.sum(t * wx2r_ref[...], axis=-1, keepdims=True) + bx2_ref[0, 0]
    msg_ref[...] = msg
    wv_ref[...] = d16 * w


def _tc_edge(g1, g2, xs, xd, ea, w1e, wsq, we2, be2, wx1, bx1, wx2r, bx2,
             *, interpret=False):
    e, dh = g1.shape
    de = ea.shape[1]
    be = 2000
    grid = (e // be,)
    wspec = lambda shape: pl.BlockSpec(shape, lambda i: tuple(0 for _ in shape))
    return pl.pallas_call(
        _edge_body,
        grid=grid,
        in_specs=[
            pl.BlockSpec((be, dh), lambda i: (i, 0)),
            pl.BlockSpec((be, dh), lambda i: (i, 0)),
            pl.BlockSpec((be, 16), lambda i: (i, 0)),
            pl.BlockSpec((be, 16), lambda i: (i, 0)),
            pl.BlockSpec((be, de), lambda i: (i, 0)),
            wspec((de, dh)),
            wspec((1, dh)),
            wspec((dh, dh)),
            wspec((1, dh)),
            wspec((dh, dh)),
            wspec((1, dh)),
            wspec((1, dh)),
            wspec((1, 1)),
        ],
        out_specs=[
            pl.BlockSpec((be, dh), lambda i: (i, 0)),
            pl.BlockSpec((be, 16), lambda i: (i, 0)),
        ],
        out_shape=[
            jax.ShapeDtypeStruct((e, dh), jnp.float32),
            jax.ShapeDtypeStruct((e, 16), jnp.float32),
        ],
        interpret=interpret,
    )(g1, g2, xs, xd, ea, w1e, wsq,
      we2.astype(jnp.bfloat16), be2, wx1.astype(jnp.bfloat16), bx1, wx2r, bx2)


def _sc_scatter(msg, wv, dst, src, n):
    e, dm = msg.shape
    pt = e // NW
    G = 80
    nchunk = pt // G
    rz = n // NS
    ZB = 25
    nz = rz // ZB
    assert pt % G == 0 and rz % ZB == 0

    mesh = plsc.VectorSubcoreMesh(core_axis_name="c", subcore_axis_name="s",
                                  num_cores=NC, num_subcores=NS)

    @functools.partial(
        pl.kernel,
        out_type=(
            jax.ShapeDtypeStruct((NC * n, dm), jnp.float32),
            jax.ShapeDtypeStruct((NC * n, 16), jnp.float32),
            jax.ShapeDtypeStruct((NC * n, 16), jnp.float32),
        ),
        mesh=mesh,
        scratch_types=[
            pltpu.VMEM((2, G), jnp.int32),
            pltpu.VMEM((2, G), jnp.int32),
            pltpu.VMEM((2, G, dm), jnp.float32),
            pltpu.VMEM((2, G, 16), jnp.float32),
            pltpu.VMEM((G, 16), jnp.float32),
            pltpu.VMEM((ZB, dm), jnp.float32),
            pltpu.VMEM((ZB, 16), jnp.float32),
            pltpu.VMEM_SHARED((n, dm), jnp.float32),
            pltpu.VMEM_SHARED((n, 16), jnp.float32),
            pltpu.VMEM_SHARED((n, 16), jnp.float32),
            pltpu.SemaphoreType.DMA,
        ],
        compiler_params=pltpu.CompilerParams(use_tc_tiling_on_sc=False),
    )
    def scatter_kernel(msg_hbm, wv_hbm, dst_hbm, src_hbm,
                       pd_hbm, pc_hbm, ps_hbm,
                       didx_v, sidx_v, msg_v, wv_v, ones_v, zbd_v, zbs_v,
                       acc_d, acc_c, acc_s, sem_l):
        cid = lax.axis_index("c")
        sid = lax.axis_index("s")
        wid = sid * NC + cid
        base = wid * pt
        r0 = sid * rz

        def zrow_d(i, carry):
            r = i // (dm // 16)
            c = (i % (dm // 16)) * 16
            zbd_v[r, pl.ds(c, 16)] = jnp.zeros((16,), jnp.float32)
            return carry
        lax.fori_loop(0, ZB * (dm // 16), zrow_d, 0)

        lanes = lax.iota(jnp.int32, 16)
        one_row = jnp.where(lanes == 0, 1.0, 0.0).astype(jnp.float32)

        def zrow_s(i, carry):
            zbs_v[i, :] = jnp.zeros((16,), jnp.float32)
            return carry
        lax.fori_loop(0, ZB, zrow_s, 0)

        def orow(i, carry):
            ones_v[i, :] = one_row
            return carry
        lax.fori_loop(0, G, orow, 0)

        def zinit(j, carry):
            pltpu.sync_copy(zbd_v, acc_d.at[pl.ds(r0 + j * ZB, ZB)])
            pltpu.sync_copy(zbs_v, acc_c.at[pl.ds(r0 + j * ZB, ZB)])
            pltpu.sync_copy(zbs_v, acc_s.at[pl.ds(r0 + j * ZB, ZB)])
            return carry
        lax.fori_loop(0, nz, zinit, 0)
        plsc.subcore_barrier()

        def fire_loads(c, b):
            off = base + c * G
            pltpu.async_copy(dst_hbm.at[pl.ds(off, G)], didx_v.at[b], sem_l)
            pltpu.async_copy(src_hbm.at[pl.ds(off, G)], sidx_v.at[b], sem_l)
            pltpu.async_copy(msg_hbm.at[pl.ds(off, G)], msg_v.at[b], sem_l)
            pltpu.async_copy(wv_hbm.at[pl.ds(off, G)], wv_v.at[b], sem_l)

        def drain_loads(b):
            pltpu.make_async_copy(dst_hbm.at[pl.ds(base, G)], didx_v.at[b], sem_l).wait()
            pltpu.make_async_copy(src_hbm.at[pl.ds(base, G)], sidx_v.at[b], sem_l).wait()
            pltpu.make_async_copy(msg_hbm.at[pl.ds(base, G)], msg_v.at[b], sem_l).wait()
            pltpu.make_async_copy(wv_hbm.at[pl.ds(base, G)], wv_v.at[b], sem_l).wait()

        fire_loads(0, 0)

        def chunk(c, carry):
            b = c % 2

            @pl.when(c + 1 < nchunk)
            def _():
                fire_loads(c + 1, 1 - b)

            drain_loads(b)
            pltpu.sync_copy(msg_v.at[b], acc_d.at[didx_v.at[b]], add=True)
            pltpu.sync_copy(ones_v, acc_c.at[didx_v.at[b]], add=True)
            pltpu.sync_copy(wv_v.at[b], acc_s.at[sidx_v.at[b]], add=True)
            return carry

        lax.fori_loop(0, nchunk, chunk, 0)
        plsc.subcore_barrier()

        def dump(j, carry):
            rr = r0 + j * ZB
            pltpu.sync_copy(acc_d.at[pl.ds(rr, ZB)], zbd_v)
            pltpu.sync_copy(zbd_v, pd_hbm.at[pl.ds(cid * n + rr, ZB)])
            pltpu.sync_copy(acc_c.at[pl.ds(rr, ZB)], zbs_v)
            pltpu.sync_copy(zbs_v, pc_hbm.at[pl.ds(cid * n + rr, ZB)])
            pltpu.sync_copy(acc_s.at[pl.ds(rr, ZB)], zbs_v)
            pltpu.sync_copy(zbs_v, ps_hbm.at[pl.ds(cid * n + rr, ZB)])
            return carry
        lax.fori_loop(0, nz, dump, 0)

    return scatter_kernel(msg, wv, dst, src)


def _post_body(pd0_ref, pd1_ref, pc0_ref, pc1_ref, ps0_ref, ps1_ref,
               h_ref, x16_ref,
               wh1a_ref, wh1b_ref, bh1_ref, wh2_ref, bh2_ref,
               lng_ref, lnb_ref, hn_ref, xn_ref):
    agg_s = pd0_ref[0] + pd1_ref[0]
    pc = pc0_ref[0] + pc1_ref[0]
    cnt = jnp.maximum(pc[:, 0:1], 1.0)
    agg = agg_s / cnt
    hb = h_ref[...]
    t = _silu(jnp.dot(hb, wh1a_ref[...], preferred_element_type=jnp.float32)
              + jnp.dot(agg, wh1b_ref[...], preferred_element_type=jnp.float32)
              + bh1_ref[...])
    ph = jnp.dot(t, wh2_ref[...], preferred_element_type=jnp.float32) + bh2_ref[...]
    pre = hb + ph
    mu = jnp.mean(pre, axis=-1, keepdims=True)
    var = jnp.mean((pre - mu) ** 2, axis=-1, keepdims=True)
    hn_ref[...] = (pre - mu) * jax.lax.rsqrt(var + 1e-5) * lng_ref[...] + lnb_ref[...]
    ps = ps0_ref[0] + ps1_ref[0]
    xn_ref[...] = x16_ref[...] + ps / cnt


def _tc_post(pd, pc, ps, h, x16, wh1a, wh1b, bh1, wh2, bh2, lng, lnb,
             *, interpret=False):
    n, dh = h.shape
    dm = pd.shape[1]
    bn = 2000
    grid = (n // bn,)
    wspec = lambda shape: pl.BlockSpec(shape, lambda i: tuple(0 for _ in shape))
    return pl.pallas_call(
        _post_body,
        grid=grid,
        in_specs=[
            pl.BlockSpec((1, bn, dm), lambda i: (0, i, 0)),
            pl.BlockSpec((1, bn, dm), lambda i: (1, i, 0)),
            pl.BlockSpec((1, bn, 16), lambda i: (0, i, 0)),
            pl.BlockSpec((1, bn, 16), lambda i: (1, i, 0)),
            pl.BlockSpec((1, bn, 16), lambda i: (0, i, 0)),
            pl.BlockSpec((1, bn, 16), lambda i: (1, i, 0)),
            pl.BlockSpec((bn, dh), lambda i: (i, 0)),
            pl.BlockSpec((bn, 16), lambda i: (i, 0)),
            wspec((dh, dh)),
            wspec((dh, dh)),
            wspec((1, dh)),
            wspec((dh, dh)),
            wspec((1, dh)),
            wspec((1, dh)),
            wspec((1, dh)),
        ],
        out_specs=[
            pl.BlockSpec((bn, dh), lambda i: (i, 0)),
            pl.BlockSpec((bn, 16), lambda i: (i, 0)),
        ],
        out_shape=[
            jax.ShapeDtypeStruct((n, dh), jnp.float32),
            jax.ShapeDtypeStruct((n, 16), jnp.float32),
        ],
        interpret=interpret,
    )(pd.reshape(2, n, dm), pd.reshape(2, n, dm),
      pc.reshape(2, n, 16), pc.reshape(2, n, 16),
      ps.reshape(2, n, 16), ps.reshape(2, n, 16), h, x16,
      wh1a, wh1b, bh1, wh2, bh2, lng, lnb)


def kernel(h, x, edge_index, edge_attr, W_e1, b_e1, W_e2, b_e2,
           W_h1, b_h1, W_h2, b_h2, W_x1, b_x1, W_x2, b_x2, ln_g, ln_b):
    n, dh = h.shape
    e = edge_index.shape[1]
    de = edge_attr.shape[1]
    dm = W_e2.shape[0]

    src = edge_index[0]
    dst = edge_index[1]
    x16 = jnp.pad(x, ((0, 0), (0, 16 - x.shape[1])))

    w1s = W_e1[:dh]
    w1d = W_e1[dh:2 * dh]
    wsq = W_e1[2 * dh:2 * dh + 1]
    w1e = W_e1[2 * dh + 1:]

    hs, hd = _tc_pre(h, w1s, w1d, b_e1.reshape(1, dm))
    g1, g2, xs, xd = _sc_gather(hs, hd, x16, src, dst)
    msg, wv = _tc_edge(g1, g2, xs, xd, edge_attr,
                       w1e, wsq, W_e2, b_e2.reshape(1, dm),
                       W_x1, b_x1.reshape(1, dm),
                       W_x2.reshape(1, dm), b_x2.reshape(1, 1))
    pd, pc, ps = _sc_scatter(msg, wv, dst, src, n)
    hn, xn16 = _tc_post(pd, pc, ps, h, x16,
                        W_h1[:dh], W_h1[dh:], b_h1.reshape(1, dh),
                        W_h2, b_h2.reshape(1, dh),
                        ln_g.reshape(1, dh), ln_b.reshape(1, dh))
    return (hn, xn16[:, :x.shape[1]])


# speedup vs baseline: 1.9505x; 1.0217x over previous
import functools

import jax
import jax.numpy as jnp
from jax import lax
from jax.experimental import pallas as pl
from jax.experimental.pallas import tpu as pltpu
from jax.experimental.pallas import tpu_sc as plsc

NC = 2
NS = 16
NW = NC * NS


def _silu(v):
    return v * jax.nn.sigmoid(v)


def _pre_body(h_ref, w1s_ref, w1d_ref, be1_ref, hs_ref, hd_ref):
    hb = h_ref[...]
    hs_ref[...] = jnp.dot(hb, w1s_ref[...], preferred_element_type=jnp.float32)
    hd_ref[...] = (jnp.dot(hb, w1d_ref[...], preferred_element_type=jnp.float32)
                   + be1_ref[...])


def _tc_pre(h, w1s, w1d, be1, *, interpret=False):
    n, dh = h.shape
    bn = 2000
    grid = (n // bn,)
    return pl.pallas_call(
        _pre_body,
        grid=grid,
        in_specs=[
            pl.BlockSpec((bn, dh), lambda i: (i, 0)),
            pl.BlockSpec((dh, dh), lambda i: (0, 0)),
            pl.BlockSpec((dh, dh), lambda i: (0, 0)),
            pl.BlockSpec((1, dh), lambda i: (0, 0)),
        ],
        out_specs=[
            pl.BlockSpec((bn, dh), lambda i: (i, 0)),
            pl.BlockSpec((bn, dh), lambda i: (i, 0)),
        ],
        out_shape=[
            jax.ShapeDtypeStruct((n, dh), jnp.float32),
            jax.ShapeDtypeStruct((n, dh), jnp.float32),
        ],
        interpret=interpret,
    )(h, w1s, w1d, be1)


def _sc_gather(hs, hd, x16, src, dst):
    n, dh = hs.shape
    e = src.shape[0]
    pt = e // NW
    G = 400
    SUB = 80
    nsub = G // SUB
    nchunk = pt // G
    assert pt % G == 0 and G % SUB == 0

    mesh = plsc.VectorSubcoreMesh(core_axis_name="c", subcore_axis_name="s",
                                  num_cores=NC, num_subcores=NS)

    @functools.partial(
        pl.kernel,
        out_type=(
            jax.ShapeDtypeStruct((e, dh), jnp.float32),
            jax.ShapeDtypeStruct((e, dh), jnp.float32),
            jax.ShapeDtypeStruct((e, 16), jnp.float32),
            jax.ShapeDtypeStruct((e, 16), jnp.float32),
        ),
        mesh=mesh,
        scratch_types=[
            pltpu.VMEM((G,), jnp.int32),
            pltpu.VMEM((G,), jnp.int32),
            pltpu.VMEM((G, dh), jnp.float32),
            pltpu.VMEM((G, dh), jnp.float32),
            pltpu.VMEM((G, 16), jnp.float32),
            pltpu.VMEM((G, 16), jnp.float32),
            pltpu.SemaphoreType.DMA,
        ],
        compiler_params=pltpu.CompilerParams(use_tc_tiling_on_sc=False),
    )
    def gather_kernel(hs_hbm, hd_hbm, x_hbm, src_hbm, dst_hbm,
                      g1_hbm, g2_hbm, xs_hbm, xd_hbm,
                      sidx_v, didx_v, g1_v, g2_v, xs_v, xd_v, sem):
        wid = lax.axis_index("s") * NC + lax.axis_index("c")
        base = wid * pt

        def chunk(c, carry):
            off = base + c * G
            pltpu.sync_copy(src_hbm.at[pl.ds(off, G)], sidx_v)
            pltpu.sync_copy(dst_hbm.at[pl.ds(off, G)], didx_v)
            cps = []
            for s in range(nsub):
                si = sidx_v.at[pl.ds(s * SUB, SUB)]
                di = didx_v.at[pl.ds(s * SUB, SUB)]
                cps.append(pltpu.async_copy(
                    hs_hbm.at[si], g1_v.at[pl.ds(s * SUB, SUB)], sem))
                cps.append(pltpu.async_copy(
                    hd_hbm.at[di], g2_v.at[pl.ds(s * SUB, SUB)], sem))
                cps.append(pltpu.async_copy(
                    x_hbm.at[si], xs_v.at[pl.ds(s * SUB, SUB)], sem))
                cps.append(pltpu.async_copy(
                    x_hbm.at[di], xd_v.at[pl.ds(s * SUB, SUB)], sem))
            for cp in cps:
                cp.wait()
            pltpu.sync_copy(g1_v, g1_hbm.at[pl.ds(off, G)])
            pltpu.sync_copy(g2_v, g2_hbm.at[pl.ds(off, G)])
            pltpu.sync_copy(xs_v, xs_hbm.at[pl.ds(off, G)])
            pltpu.sync_copy(xd_v, xd_hbm.at[pl.ds(off, G)])
            return carry

        lax.fori_loop(0, nchunk, chunk, 0)

    return gather_kernel(hs, hd, x16, src, dst)


def _edge_body(g1_ref, g2_ref, xs_ref, xd_ref, ea_ref,
               w1e_ref, wsq_ref, we2_ref, be2_ref,
               wx1_ref, bx1_ref, wx2r_ref, bx2_ref,
               msg_ref, wv_ref):
    d16 = xs_ref[...] - xd_ref[...]
    sq = jnp.sum(d16 * d16, axis=-1, keepdims=True)
    pre = (g1_ref[...] + g2_ref[...]
           + sq * wsq_ref[...]
           + jnp.dot(ea_ref[...].astype(jnp.bfloat16), w1e_ref[...],
                     preferred_element_type=jnp.float32))
    m1 = _silu(pre).astype(jnp.bfloat16)
    msg = _silu(jnp.dot(m1, we2_ref[...], preferred_element_type=jnp.float32)
                + be2_ref[...])
    msgb = msg.astype(jnp.bfloat16)
    t = _silu(jnp.dot(msgb, wx1_ref[...], preferred_element_type=jnp.float32)
              + bx1_ref[...])
    w = jnp.sum(t * wx2r_ref[...], axis=-1, keepdims=True) + bx2_ref[0, 0]
    msg_ref[...] = msg
    wv_ref[...] = d16 * w


def _tc_edge(g1, g2, xs, xd, ea, w1e, wsq, we2, be2, wx1, bx1, wx2r, bx2,
             *, interpret=False):
    e, dh = g1.shape
    de = ea.shape[1]
    be = 4000
    grid = (e // be,)
    wspec = lambda shape: pl.BlockSpec(shape, lambda i: tuple(0 for _ in shape))
    return pl.pallas_call(
        _edge_body,
        grid=grid,
        in_specs=[
            pl.BlockSpec((be, dh), lambda i: (i, 0)),
            pl.BlockSpec((be, dh), lambda i: (i, 0)),
            pl.BlockSpec((be, 16), lambda i: (i, 0)),
            pl.BlockSpec((be, 16), lambda i: (i, 0)),
            pl.BlockSpec((be, de), lambda i: (i, 0)),
            wspec((de, dh)),
            wspec((1, dh)),
            wspec((dh, dh)),
            wspec((1, dh)),
            wspec((dh, dh)),
            wspec((1, dh)),
            wspec((1, dh)),
            wspec((1, 1)),
        ],
        out_specs=[
            pl.BlockSpec((be, dh), lambda i: (i, 0)),
            pl.BlockSpec((be, 16), lambda i: (i, 0)),
        ],
        out_shape=[
            jax.ShapeDtypeStruct((e, dh), jnp.float32),
            jax.ShapeDtypeStruct((e, 16), jnp.float32),
        ],
        interpret=interpret,
    )(g1, g2, xs, xd, ea, w1e.astype(jnp.bfloat16), wsq,
      we2.astype(jnp.bfloat16), be2, wx1.astype(jnp.bfloat16), bx1, wx2r, bx2)


def _sc_scatter(msg, wv, dst, src, n):
    e, dm = msg.shape
    pt = e // NW
    G = 80
    nchunk = pt // G
    rz = n // NS
    ZB = 25
    nz = rz // ZB
    assert pt % G == 0 and rz % ZB == 0

    mesh = plsc.VectorSubcoreMesh(core_axis_name="c", subcore_axis_name="s",
                                  num_cores=NC, num_subcores=NS)

    @functools.partial(
        pl.kernel,
        out_type=(
            jax.ShapeDtypeStruct((NC * n, dm), jnp.float32),
            jax.ShapeDtypeStruct((NC * n, 16), jnp.float32),
            jax.ShapeDtypeStruct((NC * n, 16), jnp.float32),
        ),
        mesh=mesh,
        scratch_types=[
            pltpu.VMEM((2, G), jnp.int32),
            pltpu.VMEM((2, G), jnp.int32),
            pltpu.VMEM((2, G, dm), jnp.float32),
            pltpu.VMEM((2, G, 16), jnp.float32),
            pltpu.VMEM((G, 16), jnp.float32),
            pltpu.VMEM((ZB, dm), jnp.float32),
            pltpu.VMEM((ZB, 16), jnp.float32),
            pltpu.VMEM_SHARED((n, dm), jnp.float32),
            pltpu.VMEM_SHARED((n, 16), jnp.float32),
            pltpu.VMEM_SHARED((n, 16), jnp.float32),
            pltpu.SemaphoreType.DMA,
        ],
        compiler_params=pltpu.CompilerParams(use_tc_tiling_on_sc=False),
    )
    def scatter_kernel(msg_hbm, wv_hbm, dst_hbm, src_hbm,
                       pd_hbm, pc_hbm, ps_hbm,
                       didx_v, sidx_v, msg_v, wv_v, ones_v, zbd_v, zbs_v,
                       acc_d, acc_c, acc_s, sem_l):
        cid = lax.axis_index("c")
        sid = lax.axis_index("s")
        wid = sid * NC + cid
        base = wid * pt
        r0 = sid * rz

        def zrow_d(i, carry):
            r = i // (dm // 16)
            c = (i % (dm // 16)) * 16
            zbd_v[r, pl.ds(c, 16)] = jnp.zeros((16,), jnp.float32)
            return carry
        lax.fori_loop(0, ZB * (dm // 16), zrow_d, 0)

        lanes = lax.iota(jnp.int32, 16)
        one_row = jnp.where(lanes == 0, 1.0, 0.0).astype(jnp.float32)

        def zrow_s(i, carry):
            zbs_v[i, :] = jnp.zeros((16,), jnp.float32)
            return carry
        lax.fori_loop(0, ZB, zrow_s, 0)

        def orow(i, carry):
            ones_v[i, :] = one_row
            return carry
        lax.fori_loop(0, G, orow, 0)

        def zinit(j, carry):
            pltpu.sync_copy(zbd_v, acc_d.at[pl.ds(r0 + j * ZB, ZB)])
            pltpu.sync_copy(zbs_v, acc_c.at[pl.ds(r0 + j * ZB, ZB)])
            pltpu.sync_copy(zbs_v, acc_s.at[pl.ds(r0 + j * ZB, ZB)])
            return carry
        lax.fori_loop(0, nz, zinit, 0)
        plsc.subcore_barrier()

        def fire_loads(c, b):
            off = base + c * G
            pltpu.async_copy(dst_hbm.at[pl.ds(off, G)], didx_v.at[b], sem_l)
            pltpu.async_copy(src_hbm.at[pl.ds(off, G)], sidx_v.at[b], sem_l)
            pltpu.async_copy(msg_hbm.at[pl.ds(off, G)], msg_v.at[b], sem_l)
            pltpu.async_copy(wv_hbm.at[pl.ds(off, G)], wv_v.at[b], sem_l)

        def drain_loads(b):
            pltpu.make_async_copy(dst_hbm.at[pl.ds(base, G)], didx_v.at[b], sem_l).wait()
            pltpu.make_async_copy(src_hbm.at[pl.ds(base, G)], sidx_v.at[b], sem_l).wait()
            pltpu.make_async_copy(msg_hbm.at[pl.ds(base, G)], msg_v.at[b], sem_l).wait()
            pltpu.make_async_copy(wv_hbm.at[pl.ds(base, G)], wv_v.at[b], sem_l).wait()

        fire_loads(0, 0)

        def chunk(c, carry):
            b = c % 2

            @pl.when(c + 1 < nchunk)
            def _():
                fire_loads(c + 1, 1 - b)

            drain_loads(b)
            pltpu.sync_copy(msg_v.at[b], acc_d.at[didx_v.at[b]], add=True)
            pltpu.sync_copy(ones_v, acc_c.at[didx_v.at[b]], add=True)
            pltpu.sync_copy(wv_v.at[b], acc_s.at[sidx_v.at[b]], add=True)
            return carry

        lax.fori_loop(0, nchunk, chunk, 0)
        plsc.subcore_barrier()

        def dump(j, carry):
            rr = r0 + j * ZB
            pltpu.sync_copy(acc_d.at[pl.ds(rr, ZB)], zbd_v)
            pltpu.sync_copy(zbd_v, pd_hbm.at[pl.ds(cid * n + rr, ZB)])
            pltpu.sync_copy(acc_c.at[pl.ds(rr, ZB)], zbs_v)
            pltpu.sync_copy(zbs_v, pc_hbm.at[pl.ds(cid * n + rr, ZB)])
            pltpu.sync_copy(acc_s.at[pl.ds(rr, ZB)], zbs_v)
            pltpu.sync_copy(zbs_v, ps_hbm.at[pl.ds(cid * n + rr, ZB)])
            return carry
        lax.fori_loop(0, nz, dump, 0)

    return scatter_kernel(msg, wv, dst, src)


def _post_body(pd0_ref, pd1_ref, pc0_ref, pc1_ref, ps0_ref, ps1_ref,
               h_ref, x16_ref,
               wh1a_ref, wh1b_ref, bh1_ref, wh2_ref, bh2_ref,
               lng_ref, lnb_ref, hn_ref, xn_ref):
    agg_s = pd0_ref[0] + pd1_ref[0]
    pc = pc0_ref[0] + pc1_ref[0]
    cnt = jnp.maximum(pc[:, 0:1], 1.0)
    agg = agg_s / cnt
    hb = h_ref[...]
    t = _silu(jnp.dot(hb, wh1a_ref[...], preferred_element_type=jnp.float32)
              + jnp.dot(agg, wh1b_ref[...], preferred_element_type=jnp.float32)
              + bh1_ref[...])
    ph = jnp.dot(t, wh2_ref[...], preferred_element_type=jnp.float32) + bh2_ref[...]
    pre = hb + ph
    mu = jnp.mean(pre, axis=-1, keepdims=True)
    var = jnp.mean((pre - mu) ** 2, axis=-1, keepdims=True)
    hn_ref[...] = (pre - mu) * jax.lax.rsqrt(var + 1e-5) * lng_ref[...] + lnb_ref[...]
    ps = ps0_ref[0] + ps1_ref[0]
    xn_ref[...] = x16_ref[...] + ps / cnt


def _tc_post(pd, pc, ps, h, x16, wh1a, wh1b, bh1, wh2, bh2, lng, lnb,
             *, interpret=False):
    n, dh = h.shape
    dm = pd.shape[1]
    bn = 2000
    grid = (n // bn,)
    wspec = lambda shape: pl.BlockSpec(shape, lambda i: tuple(0 for _ in shape))
    return pl.pallas_call(
        _post_body,
        grid=grid,
        in_specs=[
            pl.BlockSpec((1, bn, dm), lambda i: (0, i, 0)),
            pl.BlockSpec((1, bn, dm), lambda i: (1, i, 0)),
            pl.BlockSpec((1, bn, 16), lambda i: (0, i, 0)),
            pl.BlockSpec((1, bn, 16), lambda i: (1, i, 0)),
            pl.BlockSpec((1, bn, 16), lambda i: (0, i, 0)),
            pl.BlockSpec((1, bn, 16), lambda i: (1, i, 0)),
            pl.BlockSpec((bn, dh), lambda i: (i, 0)),
            pl.BlockSpec((bn, 16), lambda i: (i, 0)),
            wspec((dh, dh)),
            wspec((dh, dh)),
            wspec((1, dh)),
            wspec((dh, dh)),
            wspec((1, dh)),
            wspec((1, dh)),
            wspec((1, dh)),
        ],
        out_specs=[
            pl.BlockSpec((bn, dh), lambda i: (i, 0)),
            pl.BlockSpec((bn, 16), lambda i: (i, 0)),
        ],
        out_shape=[
            jax.ShapeDtypeStruct((n, dh), jnp.float32),
            jax.ShapeDtypeStruct((n, 16), jnp.float32),
        ],
        interpret=interpret,
    )(pd.reshape(2, n, dm), pd.reshape(2, n, dm),
      pc.reshape(2, n, 16), pc.reshape(2, n, 16),
      ps.reshape(2, n, 16), ps.reshape(2, n, 16), h, x16,
      wh1a, wh1b, bh1, wh2, bh2, lng, lnb)


def kernel(h, x, edge_index, edge_attr, W_e1, b_e1, W_e2, b_e2,
           W_h1, b_h1, W_h2, b_h2, W_x1, b_x1, W_x2, b_x2, ln_g, ln_b):
    n, dh = h.shape
    e = edge_index.shape[1]
    de = edge_attr.shape[1]
    dm = W_e2.shape[0]

    src = edge_index[0]
    dst = edge_index[1]
    x16 = jnp.pad(x, ((0, 0), (0, 16 - x.shape[1])))

    w1s = W_e1[:dh]
    w1d = W_e1[dh:2 * dh]
    wsq = W_e1[2 * dh:2 * dh + 1]
    w1e = W_e1[2 * dh + 1:]

    hs, hd = _tc_pre(h, w1s, w1d, b_e1.reshape(1, dm))
    g1, g2, xs, xd = _sc_gather(hs, hd, x16, src, dst)
    msg, wv = _tc_edge(g1, g2, xs, xd, edge_attr,
                       w1e, wsq, W_e2, b_e2.reshape(1, dm),
                       W_x1, b_x1.reshape(1, dm),
                       W_x2.reshape(1, dm), b_x2.reshape(1, 1))
    pd, pc, ps = _sc_scatter(msg, wv, dst, src, n)
    hn, xn16 = _tc_post(pd, pc, ps, h, x16,
                        W_h1[:dh], W_h1[dh:], b_h1.reshape(1, dh),
                        W_h2, b_h2.reshape(1, dh),
                        ln_g.reshape(1, dh), ln_b.reshape(1, dh))
    return (hn, xn16[:, :x.shape[1]])
